# baseline (speedup 1.0000x reference)
import functools

import jax
import jax.numpy as jnp
from jax import lax
from jax.experimental import pallas as pl
from jax.experimental.pallas import tpu as pltpu
from jax.experimental.pallas import tpu_sc as plsc

B, C, K, D = 16384, 20, 10, 64
NC, NS = 2, 16
NW = NC * NS
RPW = B // NW
G = 16
NG = RPW // G
CCH = (G * C) // 80
NCH = (G * K) // 80

_MESH = plsc.VectorSubcoreMesh(
    core_axis_name="c", subcore_axis_name="s", num_cores=NC, num_subcores=NS
)
_SC_PARAMS = pltpu.CompilerParams(
    needs_layout_passes=False, use_tc_tiling_on_sc=True)


def _prow(v):
    return ((v >> 14) << 13) + (v & 8191)


def _poff(v):
    return (v & 8192) >> 7


def _sca_body(ctxidx_hbm, len_hbm, in_emb, sums_out,
              ctxidx_v, len_v, ctx_buf, sums_v, pci_v, coff_v, sem):
    wid = lax.axis_index("s") * NC + lax.axis_index("c")
    pltpu.sync_copy(ctxidx_hbm.at[wid], ctxidx_v)
    pltpu.sync_copy(len_hbm.at[wid], len_v)

    def group_body(g, carry):
        for i in range(CCH):
            for t in range(5):
                v = ctxidx_v[g * CCH + i, pl.ds(t * 16, 16)]
                pci_v[pl.ds(i * 80 + t * 16, 16)] = _prow(v)
                coff_v[pl.ds(i * 80 + t * 16, 16)] = _poff(v)
        cps = []
        for i in range(CCH):
            cp = pltpu.make_async_copy(
                in_emb.at[pci_v.at[pl.ds(i * 80, 80)]],
                ctx_buf.at[pl.ds(i * 80, 80)], sem)
            cp.start()
            cps.append(cp)
        for cp in cps:
            cp.wait()

        def row_body(r, carry2):
            oa = coff_v[pl.ds(r * C, 16)]
            ob = coff_v[pl.ds(r * C + 16, 16)]

            def coff(c):
                return oa[c] if c < 16 else ob[c - 16]

            o = coff(0)
            accs = [ctx_buf[r * C, pl.ds(o + j * 16, 16)] for j in range(4)]
            for c in range(1, C):
                o = coff(c)
                for j in range(4):
                    accs[j] = accs[j] + ctx_buf[r * C + c,
                                                pl.ds(o + j * 16, 16)]
            row = g * G + r
            for j in range(4):
                sums_v[row, pl.ds(j * 16, 16)] = accs[j]
            return carry2

        lax.fori_loop(0, G, row_body, 0)
        return carry

    lax.fori_loop(0, NG, group_body, 0)
    pltpu.sync_copy(sums_v, sums_out.at[wid])


_sc_sums = functools.partial(
    pl.kernel,
    out_type=jax.ShapeDtypeStruct((NW, RPW, D), jnp.float32),
    mesh=_MESH,
    compiler_params=_SC_PARAMS,
    scratch_types=[
        pltpu.VMEM((NG * CCH, 80), jnp.int32),
        pltpu.VMEM((NG, G), jnp.int32),
        pltpu.VMEM((G * C, 2 * D), jnp.float32),
        pltpu.VMEM((RPW, D), jnp.float32),
        pltpu.VMEM((G * C,), jnp.int32),
        pltpu.VMEM((G * C + 16,), jnp.int32),
        pltpu.SemaphoreType.DMA,
    ],
)(_sca_body)


def _scb_body(negidx_hbm, posidx_hbm, len_hbm, sums_hbm, out_emb,
              pos_out, neg_out,
              negidx_v, posidx_v, len_v, sums_v, neg_buf, pos_buf,
              pos_s, neg_s, pni_v, noff_v, ppi_v, poff_v, sem):
    wid = lax.axis_index("s") * NC + lax.axis_index("c")
    pltpu.sync_copy(negidx_hbm.at[wid], negidx_v)
    pltpu.sync_copy(posidx_hbm.at[wid], posidx_v)
    pltpu.sync_copy(len_hbm.at[wid], len_v)
    pltpu.sync_copy(sums_hbm.at[wid], sums_v)

    iota = lax.iota(jnp.int32, 16)

    def group_body(g, carry):
        for i in range(NCH):
            for t in range(5):
                v = negidx_v[g * NCH + i, pl.ds(t * 16, 16)]
                pni_v[pl.ds(i * 80 + t * 16, 16)] = _prow(v)
                noff_v[pl.ds(i * 80 + t * 16, 16)] = _poff(v)
        v = posidx_v[g, :]
        ppi_v[...] = _prow(v)
        poff_v[pl.ds(0, 16)] = _poff(v)

        cps = []
        for i in range(NCH):
            cp = pltpu.make_async_copy(
                out_emb.at[pni_v.at[pl.ds(i * 80, 80)]],
                neg_buf.at[pl.ds(i * 80, 80)], sem)
            cp.start()
            cps.append(cp)
        cp = pltpu.make_async_copy(out_emb.at[ppi_v], pos_buf, sem)
        cp.start()
        cps.append(cp)
        for cp in cps:
            cp.wait()

        len_f = len_v[g, :].astype(jnp.float32)
        recip = 1.0 / jnp.maximum(len_f, 1.0)

        def row_body(r, scores):
            onehot = iota == r
            on_ = noff_v[pl.ds(r * K, 16)]
            op_ = poff_v[pl.ds(r, 16)][0]
            row = g * G + r
            accs = [sums_v[row, pl.ds(j * 16, 16)] for j in range(4)]
            new_scores = []
            part = accs[0] * pos_buf[r, pl.ds(op_, 16)]
            for j in range(1, 4):
                part = part + accs[j] * pos_buf[r, pl.ds(op_ + j * 16, 16)]
            new_scores.append(jnp.where(onehot, jnp.sum(part), scores[0]))
            for k in range(K):
                o = on_[k]
                part = accs[0] * neg_buf[r * K + k, pl.ds(o, 16)]
                for j in range(1, 4):
                    part = part + accs[j] * neg_buf[r * K + k,
                                                    pl.ds(o + j * 16, 16)]
                new_scores.append(
                    jnp.where(onehot, jnp.sum(part), scores[1 + k]))
            return tuple(new_scores)

        scores0 = tuple(jnp.zeros((16,), jnp.float32) for _ in range(K + 1))
        scores = lax.fori_loop(0, G, row_body, scores0)
        pos_s[pl.ds(g * G, G)] = scores[0] * recip
        for k in range(K):
            neg_s[k, pl.ds(g * G, G)] = scores[1 + k] * recip
        return carry

    lax.fori_loop(0, NG, group_body, 0)

    pltpu.sync_copy(pos_s, pos_out.at[wid])
    pltpu.sync_copy(neg_s, neg_out.at[wid])


_sc_scores = functools.partial(
    pl.kernel,
    out_type=[
        jax.ShapeDtypeStruct((NW, RPW), jnp.float32),
        jax.ShapeDtypeStruct((NW, K, RPW), jnp.float32),
    ],
    mesh=_MESH,
    compiler_params=_SC_PARAMS,
    scratch_types=[
        pltpu.VMEM((NG * NCH, 80), jnp.int32),
        pltpu.VMEM((NG, G), jnp.int32),
        pltpu.VMEM((NG, G), jnp.int32),
        pltpu.VMEM((RPW, D), jnp.float32),
        pltpu.VMEM((G * K, 2 * D), jnp.float32),
        pltpu.VMEM((G, 2 * D), jnp.float32),
        pltpu.VMEM((RPW,), jnp.float32),
        pltpu.VMEM((K, RPW), jnp.float32),
        pltpu.VMEM((G * K,), jnp.int32),
        pltpu.VMEM((G * K + 16,), jnp.int32),
        pltpu.VMEM((G,), jnp.int32),
        pltpu.VMEM((2 * G,), jnp.int32),
        pltpu.SemaphoreType.DMA,
    ],
)(_scb_body)


_TCH = 16384
_THF = _TCH // 2


def _tr_body(t_ref, o_ref):
    eye = jnp.eye(64, dtype=jnp.float32)
    xt = lax.dot_general(t_ref[...], eye, (((0,), (0,)), ((), ())),
                         preferred_element_type=jnp.float32)
    o_ref[...] = jnp.concatenate([xt[0:_THF], xt[_THF:_TCH]], axis=1)


_tr = pl.pallas_call(
    _tr_body,
    grid=(62,),
    in_specs=[pl.BlockSpec((64, _TCH), lambda i: (0, i))],
    out_specs=pl.BlockSpec((_THF, 128), lambda i: (i, 0)),
    out_shape=jax.ShapeDtypeStruct((61 * _THF + 576, 2 * D), jnp.float32),
)


def _loss_body(pos_ref, neg_ref, out_ref):
    p = pos_ref[...]
    n = neg_ref[...]

    def logsig(x):
        return jnp.minimum(x, 0.0) - jnp.log1p(jnp.exp(-jnp.abs(x)))

    tot = jnp.sum(logsig(p)) + jnp.sum(logsig(-n))
    out_ref[0, 0] = -tot / B


_loss = pl.pallas_call(
    _loss_body,
    out_shape=jax.ShapeDtypeStruct((1, 1), jnp.float32),
    out_specs=pl.BlockSpec(memory_space=pltpu.SMEM),
)


def kernel(contexts, lengths, targets, neg_samples, in_embed, out_embed):
    in2 = _tr(in_embed.T)
    out2 = _tr(out_embed.T)
    ctx_idx = contexts.reshape(NW, NG * CCH, 80)
    neg_idx = neg_samples.reshape(NW, NG * NCH, 80)
    pos_idx = targets.reshape(NW, NG, G)
    len_r = lengths.reshape(NW, NG, G)
    sums = _sc_sums(ctx_idx, len_r, in2)
    pos_sc, neg_sc = _sc_scores(neg_idx, pos_idx, len_r, sums, out2)
    loss = _loss(pos_sc.reshape(128, 128), neg_sc.reshape(1280, 128))
    return loss[0, 0]


# speedup vs baseline: 1.0841x; 1.0841x over previous
import functools

import jax
import jax.numpy as jnp
from jax import lax
from jax.experimental import pallas as pl
from jax.experimental.pallas import tpu as pltpu
from jax.experimental.pallas import tpu_sc as plsc

B, C, K, D = 16384, 20, 10, 64
NC, NS = 2, 16
NW = NC * NS
RPW = B // NW
G = 16
NG = RPW // G
CCH = (G * C) // 80
NCH = (G * K) // 80
GA = 4
NGA = RPW // GA
CCHA = (GA * C) // 80

_MESH = plsc.VectorSubcoreMesh(
    core_axis_name="c", subcore_axis_name="s", num_cores=NC, num_subcores=NS
)
_SC_PARAMS = pltpu.CompilerParams(
    needs_layout_passes=False, use_tc_tiling_on_sc=True)


def _prow(v):
    return ((v >> 14) << 13) + (v & 8191)


def _poff(v):
    return (v & 8192) >> 7


def _sca_body(ctxidx_hbm, len_hbm, in_emb, sums_out,
              ctxidx_v, len_v, ctx_buf0, ctx_buf1, sums_v,
              pci_v0, pci_v1, coff_v0, coff_v1, sem0, sem1):
    wid = lax.axis_index("s") * NC + lax.axis_index("c")
    pltpu.sync_copy(ctxidx_hbm.at[wid], ctxidx_v)
    pltpu.sync_copy(len_hbm.at[wid], len_v)
    sems = (sem0, sem1)
    ctx_bufs = (ctx_buf0, ctx_buf1)
    pci_vs = (pci_v0, pci_v1)
    coff_vs = (coff_v0, coff_v1)

    def copies(g, s):
        return [pltpu.make_async_copy(
            in_emb.at[pci_vs[s].at[pl.ds(i * 80, 80)]],
            ctx_bufs[s].at[pl.ds(i * 80, 80)], sems[s])
            for i in range(CCHA)]

    def issue(g, s):
        for i in range(CCHA):
            for t in range(5):
                v = ctxidx_v[g * CCHA + i, pl.ds(t * 16, 16)]
                pci_vs[s][pl.ds(i * 80 + t * 16, 16)] = _prow(v)
                coff_vs[s][pl.ds(i * 80 + t * 16, 16)] = _poff(v)
        for cp in copies(g, s):
            cp.start()

    def compute(g, s):
        for cp in copies(g, s):
            cp.wait()

        def row_body(r, carry2):
            oa = coff_vs[s][pl.ds(r * C, 16)]
            ob = coff_vs[s][pl.ds(r * C + 16, 16)]

            def coff(c):
                return oa[c] if c < 16 else ob[c - 16]

            o = coff(0)
            accs = [ctx_bufs[s][r * C, pl.ds(o + j * 16, 16)]
                    for j in range(4)]
            for c in range(1, C):
                o = coff(c)
                for j in range(4):
                    accs[j] = accs[j] + ctx_bufs[s][r * C + c,
                                                   pl.ds(o + j * 16, 16)]
            row = g * GA + r
            for j in range(4):
                sums_v[row, pl.ds(j * 16, 16)] = accs[j]
            return carry2

        lax.fori_loop(0, GA, row_body, 0)

    issue(0, 0)

    def pair_body(g2, carry):
        g = g2 * 2
        issue(g + 1, 1)
        compute(g, 0)
        issue(g + 2, 0)
        compute(g + 1, 1)
        return carry

    lax.fori_loop(0, NGA // 2 - 1, pair_body, 0)
    g = NGA - 2
    issue(g + 1, 1)
    compute(g, 0)
    compute(g + 1, 1)
    pltpu.sync_copy(sums_v, sums_out.at[wid])


_sc_sums = functools.partial(
    pl.kernel,
    out_type=jax.ShapeDtypeStruct((NW, RPW, D), jnp.float32),
    mesh=_MESH,
    compiler_params=_SC_PARAMS,
    scratch_types=[
        pltpu.VMEM((NGA * CCHA, 80), jnp.int32),
        pltpu.VMEM((NG, G), jnp.int32),
        pltpu.VMEM((GA * C, 2 * D), jnp.float32),
        pltpu.VMEM((GA * C, 2 * D), jnp.float32),
        pltpu.VMEM((RPW, D), jnp.float32),
        pltpu.VMEM((GA * C,), jnp.int32),
        pltpu.VMEM((GA * C,), jnp.int32),
        pltpu.VMEM((GA * C + 16,), jnp.int32),
        pltpu.VMEM((GA * C + 16,), jnp.int32),
        pltpu.SemaphoreType.DMA,
        pltpu.SemaphoreType.DMA,
    ],
)(_sca_body)


def _scb_body(negidx_hbm, posidx_hbm, len_hbm, sums_hbm, out_emb,
              pos_out, neg_out,
              negidx_v, posidx_v, len_v, sums_v0, sums_v1, neg_buf0, neg_buf1,
              pos_buf0, pos_buf1, pos_s, neg_s, pni_v0, pni_v1,
              noff_v0, noff_v1, ppi_v0, ppi_v1, poff_v0, poff_v1,
              sem0, sem1):
    wid = lax.axis_index("s") * NC + lax.axis_index("c")
    pltpu.sync_copy(negidx_hbm.at[wid], negidx_v)
    pltpu.sync_copy(posidx_hbm.at[wid], posidx_v)
    pltpu.sync_copy(len_hbm.at[wid], len_v)

    iota = lax.iota(jnp.int32, 16)
    sems = (sem0, sem1)
    sums_vs = (sums_v0, sums_v1)
    neg_bufs = (neg_buf0, neg_buf1)
    pos_bufs = (pos_buf0, pos_buf1)
    pni_vs = (pni_v0, pni_v1)
    noff_vs = (noff_v0, noff_v1)
    ppi_vs = (ppi_v0, ppi_v1)
    poff_vs = (poff_v0, poff_v1)

    def copies(g, s):
        cps = [pltpu.make_async_copy(
            out_emb.at[pni_vs[s].at[pl.ds(i * 80, 80)]],
            neg_bufs[s].at[pl.ds(i * 80, 80)], sems[s])
            for i in range(NCH)]
        cps.append(pltpu.make_async_copy(
            out_emb.at[ppi_vs[s]], pos_bufs[s], sems[s]))
        cps.append(pltpu.make_async_copy(
            sums_hbm.at[wid, pl.ds(g * G, G)], sums_vs[s], sems[s]))
        return cps

    def issue(g, s):
        for i in range(NCH):
            for t in range(5):
                v = negidx_v[g * NCH + i, pl.ds(t * 16, 16)]
                pni_vs[s][pl.ds(i * 80 + t * 16, 16)] = _prow(v)
                noff_vs[s][pl.ds(i * 80 + t * 16, 16)] = _poff(v)
        v = posidx_v[g, :]
        ppi_vs[s][...] = _prow(v)
        poff_vs[s][pl.ds(0, 16)] = _poff(v)
        for cp in copies(g, s):
            cp.start()

    def compute(g, s):
        for cp in copies(g, s):
            cp.wait()
        len_f = len_v[g, :].astype(jnp.float32)
        recip = 1.0 / jnp.maximum(len_f, 1.0)

        def row_body(r, scores):
            onehot = iota == r
            on_ = noff_vs[s][pl.ds(r * K, 16)]
            op_ = poff_vs[s][pl.ds(r, 16)][0]
            accs = [sums_vs[s][r, pl.ds(j * 16, 16)] for j in range(4)]
            new_scores = []
            part = accs[0] * pos_bufs[s][r, pl.ds(op_, 16)]
            for j in range(1, 4):
                part = part + accs[j] * pos_bufs[s][r, pl.ds(op_ + j * 16, 16)]
            new_scores.append(jnp.where(onehot, jnp.sum(part), scores[0]))
            for k in range(K):
                o = on_[k]
                part = accs[0] * neg_bufs[s][r * K + k, pl.ds(o, 16)]
                for j in range(1, 4):
                    part = part + accs[j] * neg_bufs[s][r * K + k,
                                                       pl.ds(o + j * 16, 16)]
                new_scores.append(
                    jnp.where(onehot, jnp.sum(part), scores[1 + k]))
            return tuple(new_scores)

        scores0 = tuple(jnp.zeros((16,), jnp.float32) for _ in range(K + 1))
        scores = lax.fori_loop(0, G, row_body, scores0)
        pos_s[pl.ds(g * G, G)] = scores[0] * recip
        for k in range(K):
            neg_s[k, pl.ds(g * G, G)] = scores[1 + k] * recip

    issue(0, 0)

    def pair_body(g2, carry):
        g = g2 * 2
        issue(g + 1, 1)
        compute(g, 0)
        issue(g + 2, 0)
        compute(g + 1, 1)
        return carry

    lax.fori_loop(0, NG // 2 - 1, pair_body, 0)
    g = NG - 2
    issue(g + 1, 1)
    compute(g, 0)
    compute(g + 1, 1)

    pltpu.sync_copy(pos_s, pos_out.at[wid])
    pltpu.sync_copy(neg_s, neg_out.at[wid])


_sc_scores = functools.partial(
    pl.kernel,
    out_type=[
        jax.ShapeDtypeStruct((NW, RPW), jnp.float32),
        jax.ShapeDtypeStruct((NW, K, RPW), jnp.float32),
    ],
    mesh=_MESH,
    compiler_params=_SC_PARAMS,
    scratch_types=[
        pltpu.VMEM((NG * NCH, 80), jnp.int32),
        pltpu.VMEM((NG, G), jnp.int32),
        pltpu.VMEM((NG, G), jnp.int32),
        pltpu.VMEM((G, D), jnp.float32),
        pltpu.VMEM((G, D), jnp.float32),
        pltpu.VMEM((G * K, 2 * D), jnp.float32),
        pltpu.VMEM((G * K, 2 * D), jnp.float32),
        pltpu.VMEM((G, 2 * D), jnp.float32),
        pltpu.VMEM((G, 2 * D), jnp.float32),
        pltpu.VMEM((RPW,), jnp.float32),
        pltpu.VMEM((K, RPW), jnp.float32),
        pltpu.VMEM((G * K,), jnp.int32),
        pltpu.VMEM((G * K,), jnp.int32),
        pltpu.VMEM((G * K + 16,), jnp.int32),
        pltpu.VMEM((G * K + 16,), jnp.int32),
        pltpu.VMEM((G,), jnp.int32),
        pltpu.VMEM((G,), jnp.int32),
        pltpu.VMEM((2 * G,), jnp.int32),
        pltpu.VMEM((2 * G,), jnp.int32),
        pltpu.SemaphoreType.DMA,
        pltpu.SemaphoreType.DMA,
    ],
)(_scb_body)


_TCH = 16384
_THF = _TCH // 2


def _tr_body(t_ref, o_ref):
    xt = t_ref[...].T
    o_ref[...] = jnp.concatenate([xt[0:_THF], xt[_THF:_TCH]], axis=1)


_tr = pl.pallas_call(
    _tr_body,
    grid=(62,),
    in_specs=[pl.BlockSpec((64, _TCH), lambda i: (0, i))],
    out_specs=pl.BlockSpec((_THF, 128), lambda i: (i, 0)),
    out_shape=jax.ShapeDtypeStruct((61 * _THF + 576, 2 * D), jnp.float32),
)


def _loss_body(pos_ref, neg_ref, out_ref):
    p = pos_ref[...]
    n = neg_ref[...]

    def logsig(x):
        return jnp.minimum(x, 0.0) - jnp.log1p(jnp.exp(-jnp.abs(x)))

    tot = jnp.sum(logsig(p)) + jnp.sum(logsig(-n))
    out_ref[0, 0] = -tot / B


_loss = pl.pallas_call(
    _loss_body,
    out_shape=jax.ShapeDtypeStruct((1, 1), jnp.float32),
    out_specs=pl.BlockSpec(memory_space=pltpu.SMEM),
)


def kernel(contexts, lengths, targets, neg_samples, in_embed, out_embed):
    in2 = _tr(in_embed.T)
    out2 = _tr(out_embed.T)
    ctx_idx = contexts.reshape(NW, NG * CCH, 80)
    neg_idx = neg_samples.reshape(NW, NG * NCH, 80)
    pos_idx = targets.reshape(NW, NG, G)
    len_r = lengths.reshape(NW, NG, G)
    sums = _sc_sums(ctx_idx, len_r, in2)
    pos_sc, neg_sc = _sc_scores(neg_idx, pos_idx, len_r, sums, out2)
    loss = _loss(pos_sc.reshape(128, 128), neg_sc.reshape(1280, 128))
    return loss[0, 0]


# speedup vs baseline: 1.1230x; 1.0359x over previous
import functools

import jax
import jax.numpy as jnp
from jax import lax
from jax.experimental import pallas as pl
from jax.experimental.pallas import tpu as pltpu
from jax.experimental.pallas import tpu_sc as plsc

B, C, K, D = 16384, 20, 10, 64
NC, NS = 2, 16
NW = NC * NS
RPW = B // NW
G = 16
NG = RPW // G
CCH = (G * C) // 80
NCH = (G * K) // 80
GA = 4
NGA = RPW // GA
CCHA = (GA * C) // 80

_MESH = plsc.VectorSubcoreMesh(
    core_axis_name="c", subcore_axis_name="s", num_cores=NC, num_subcores=NS
)
_SC_PARAMS = pltpu.CompilerParams(
    needs_layout_passes=False, use_tc_tiling_on_sc=True)


def _prow(v):
    return ((v >> 15) << 14) + (v & 16383)


def _poff(v):
    return (v & 16384) >> 8


def _sca_body(ctxidx_hbm, len_hbm, in_emb, sums_out,
              ctxidx_v, len_v, ctx_buf0, ctx_buf1, sums_v,
              pci_v0, pci_v1, coff_v0, coff_v1, sem0, sem1):
    wid = lax.axis_index("s") * NC + lax.axis_index("c")
    pltpu.sync_copy(ctxidx_hbm.at[wid], ctxidx_v)
    pltpu.sync_copy(len_hbm.at[wid], len_v)
    sems = (sem0, sem1)
    ctx_bufs = (ctx_buf0, ctx_buf1)
    pci_vs = (pci_v0, pci_v1)
    coff_vs = (coff_v0, coff_v1)

    def copies(g, s):
        return [pltpu.make_async_copy(
            in_emb.at[pci_vs[s].at[pl.ds(i * 80, 80)]],
            ctx_bufs[s].at[pl.ds(i * 80, 80)], sems[s])
            for i in range(CCHA)]

    def issue(g, s):
        for i in range(CCHA):
            for t in range(5):
                v = ctxidx_v[g * CCHA + i, pl.ds(t * 16, 16)]
                pci_vs[s][pl.ds(i * 80 + t * 16, 16)] = _prow(v)
                coff_vs[s][pl.ds(i * 80 + t * 16, 16)] = _poff(v)
        for cp in copies(g, s):
            cp.start()

    def compute(g, s):
        for cp in copies(g, s):
            cp.wait()

        def row_body(r, carry2):
            oa = coff_vs[s][pl.ds(r * C, 16)]
            ob = coff_vs[s][pl.ds(r * C + 16, 16)]

            def coff(c):
                return oa[c] if c < 16 else ob[c - 16]

            o = coff(0)
            accs = [ctx_bufs[s][r * C, pl.ds(o + j * 16, 16)]
                    for j in range(4)]
            for c in range(1, C):
                o = coff(c)
                for j in range(4):
                    accs[j] = accs[j] + ctx_bufs[s][r * C + c,
                                                   pl.ds(o + j * 16, 16)]
            row = g * GA + r
            for j in range(4):
                sums_v[row, pl.ds(j * 16, 16)] = accs[j]
            return carry2

        lax.fori_loop(0, GA, row_body, 0)

    issue(0, 0)

    def pair_body(g2, carry):
        g = g2 * 2
        issue(g + 1, 1)
        compute(g, 0)
        issue(g + 2, 0)
        compute(g + 1, 1)
        return carry

    lax.fori_loop(0, NGA // 2 - 1, pair_body, 0)
    g = NGA - 2
    issue(g + 1, 1)
    compute(g, 0)
    compute(g + 1, 1)
    pltpu.sync_copy(sums_v, sums_out.at[wid])


_sc_sums = functools.partial(
    pl.kernel,
    out_type=jax.ShapeDtypeStruct((NW, RPW, D), jnp.float32),
    mesh=_MESH,
    compiler_params=_SC_PARAMS,
    scratch_types=[
        pltpu.VMEM((NGA * CCHA, 80), jnp.int32),
        pltpu.VMEM((NG, G), jnp.int32),
        pltpu.VMEM((GA * C, 2 * D), jnp.float32),
        pltpu.VMEM((GA * C, 2 * D), jnp.float32),
        pltpu.VMEM((RPW, D), jnp.float32),
        pltpu.VMEM((GA * C,), jnp.int32),
        pltpu.VMEM((GA * C,), jnp.int32),
        pltpu.VMEM((GA * C + 16,), jnp.int32),
        pltpu.VMEM((GA * C + 16,), jnp.int32),
        pltpu.SemaphoreType.DMA,
        pltpu.SemaphoreType.DMA,
    ],
)(_sca_body)


def _scb_body(negidx_hbm, posidx_hbm, len_hbm, sums_hbm, out_emb,
              pos_out, neg_out,
              negidx_v, posidx_v, len_v, sums_v0, sums_v1, neg_buf0, neg_buf1,
              pos_buf0, pos_buf1, pos_s, neg_s, pni_v0, pni_v1,
              noff_v0, noff_v1, ppi_v0, ppi_v1, poff_v0, poff_v1,
              sem0, sem1):
    wid = lax.axis_index("s") * NC + lax.axis_index("c")
    pltpu.sync_copy(negidx_hbm.at[wid], negidx_v)
    pltpu.sync_copy(posidx_hbm.at[wid], posidx_v)
    pltpu.sync_copy(len_hbm.at[wid], len_v)

    iota = lax.iota(jnp.int32, 16)
    sems = (sem0, sem1)
    sums_vs = (sums_v0, sums_v1)
    neg_bufs = (neg_buf0, neg_buf1)
    pos_bufs = (pos_buf0, pos_buf1)
    pni_vs = (pni_v0, pni_v1)
    noff_vs = (noff_v0, noff_v1)
    ppi_vs = (ppi_v0, ppi_v1)
    poff_vs = (poff_v0, poff_v1)

    def copies(g, s):
        cps = [pltpu.make_async_copy(
            out_emb.at[pni_vs[s].at[pl.ds(i * 80, 80)]],
            neg_bufs[s].at[pl.ds(i * 80, 80)], sems[s])
            for i in range(NCH)]
        cps.append(pltpu.make_async_copy(
            out_emb.at[ppi_vs[s]], pos_bufs[s], sems[s]))
        cps.append(pltpu.make_async_copy(
            sums_hbm.at[wid, pl.ds(g * G, G)], sums_vs[s], sems[s]))
        return cps

    def issue(g, s):
        for i in range(NCH):
            for t in range(5):
                v = negidx_v[g * NCH + i, pl.ds(t * 16, 16)]
                pni_vs[s][pl.ds(i * 80 + t * 16, 16)] = _prow(v)
                noff_vs[s][pl.ds(i * 80 + t * 16, 16)] = _poff(v)
        v = posidx_v[g, :]
        ppi_vs[s][...] = _prow(v)
        poff_vs[s][pl.ds(0, 16)] = _poff(v)
        for cp in copies(g, s):
            cp.start()

    def compute(g, s):
        for cp in copies(g, s):
            cp.wait()
        len_f = len_v[g, :].astype(jnp.float32)
        recip = 1.0 / jnp.maximum(len_f, 1.0)

        def row_body(r, scores):
            onehot = iota == r
            on_ = noff_vs[s][pl.ds(r * K, 16)]
            op_ = poff_vs[s][pl.ds(r, 16)][0]
            accs = [sums_vs[s][r, pl.ds(j * 16, 16)] for j in range(4)]
            new_scores = []
            part = accs[0] * pos_bufs[s][r, pl.ds(op_, 16)]
            for j in range(1, 4):
                part = part + accs[j] * pos_bufs[s][r, pl.ds(op_ + j * 16, 16)]
            new_scores.append(jnp.where(onehot, jnp.sum(part), scores[0]))
            for k in range(K):
                o = on_[k]
                part = accs[0] * neg_bufs[s][r * K + k, pl.ds(o, 16)]
                for j in range(1, 4):
                    part = part + accs[j] * neg_bufs[s][r * K + k,
                                                       pl.ds(o + j * 16, 16)]
                new_scores.append(
                    jnp.where(onehot, jnp.sum(part), scores[1 + k]))
            return tuple(new_scores)

        scores0 = tuple(jnp.zeros((16,), jnp.float32) for _ in range(K + 1))
        scores = lax.fori_loop(0, G, row_body, scores0)
        pos_s[pl.ds(g * G, G)] = scores[0] * recip
        for k in range(K):
            neg_s[k, pl.ds(g * G, G)] = scores[1 + k] * recip

    issue(0, 0)

    def pair_body(g2, carry):
        g = g2 * 2
        issue(g + 1, 1)
        compute(g, 0)
        issue(g + 2, 0)
        compute(g + 1, 1)
        return carry

    lax.fori_loop(0, NG // 2 - 1, pair_body, 0)
    g = NG - 2
    issue(g + 1, 1)
    compute(g, 0)
    compute(g + 1, 1)

    pltpu.sync_copy(pos_s, pos_out.at[wid])
    pltpu.sync_copy(neg_s, neg_out.at[wid])


_sc_scores = functools.partial(
    pl.kernel,
    out_type=[
        jax.ShapeDtypeStruct((NW, RPW), jnp.float32),
        jax.ShapeDtypeStruct((NW, K, RPW), jnp.float32),
    ],
    mesh=_MESH,
    compiler_params=_SC_PARAMS,
    scratch_types=[
        pltpu.VMEM((NG * NCH, 80), jnp.int32),
        pltpu.VMEM((NG, G), jnp.int32),
        pltpu.VMEM((NG, G), jnp.int32),
        pltpu.VMEM((G, D), jnp.float32),
        pltpu.VMEM((G, D), jnp.float32),
        pltpu.VMEM((G * K, 2 * D), jnp.float32),
        pltpu.VMEM((G * K, 2 * D), jnp.float32),
        pltpu.VMEM((G, 2 * D), jnp.float32),
        pltpu.VMEM((G, 2 * D), jnp.float32),
        pltpu.VMEM((RPW,), jnp.float32),
        pltpu.VMEM((K, RPW), jnp.float32),
        pltpu.VMEM((G * K,), jnp.int32),
        pltpu.VMEM((G * K,), jnp.int32),
        pltpu.VMEM((G * K + 16,), jnp.int32),
        pltpu.VMEM((G * K + 16,), jnp.int32),
        pltpu.VMEM((G,), jnp.int32),
        pltpu.VMEM((G,), jnp.int32),
        pltpu.VMEM((2 * G,), jnp.int32),
        pltpu.VMEM((2 * G,), jnp.int32),
        pltpu.SemaphoreType.DMA,
        pltpu.SemaphoreType.DMA,
    ],
)(_scb_body)


_TCH = 32768
_THF = _TCH // 2


def _tr_body(t_ref, o_ref):
    xt = t_ref[...].T
    o_ref[...] = jnp.concatenate([xt[0:_THF], xt[_THF:_TCH]], axis=1)


_tr = pl.pallas_call(
    _tr_body,
    grid=(31,),
    in_specs=[pl.BlockSpec((64, _TCH), lambda i: (0, i))],
    out_specs=pl.BlockSpec((_THF, 128), lambda i: (i, 0)),
    out_shape=jax.ShapeDtypeStruct((30 * _THF + 17088, 2 * D), jnp.float32),
)


def _loss_body(pos_ref, neg_ref, out_ref):
    p = pos_ref[...]
    n = neg_ref[...]

    def logsig(x):
        return jnp.minimum(x, 0.0) - jnp.log1p(jnp.exp(-jnp.abs(x)))

    tot = jnp.sum(logsig(p)) + jnp.sum(logsig(-n))
    out_ref[0, 0] = -tot / B


_loss = pl.pallas_call(
    _loss_body,
    out_shape=jax.ShapeDtypeStruct((1, 1), jnp.float32),
    out_specs=pl.BlockSpec(memory_space=pltpu.SMEM),
)


def kernel(contexts, lengths, targets, neg_samples, in_embed, out_embed):
    in2 = _tr(in_embed.T)
    out2 = _tr(out_embed.T)
    ctx_idx = contexts.reshape(NW, NG * CCH, 80)
    neg_idx = neg_samples.reshape(NW, NG * NCH, 80)
    pos_idx = targets.reshape(NW, NG, G)
    len_r = lengths.reshape(NW, NG, G)
    sums = _sc_sums(ctx_idx, len_r, in2)
    pos_sc, neg_sc = _sc_scores(neg_idx, pos_idx, len_r, sums, out2)
    loss = _loss(pos_sc.reshape(128, 128), neg_sc.reshape(1280, 128))
    return loss[0, 0]
